# Initial kernel scaffold; baseline (speedup 1.0000x reference)
import jax
import jax.numpy as jnp
from jax.experimental import pallas as pl


def kernel(x_a, x_v, W, a):
    raise NotImplementedError("write your pallas kernel here")


# speedup vs baseline: 14.5549x; 14.5549x over previous
import functools

import jax
import jax.numpy as jnp
from jax.experimental import pallas as pl
from jax.experimental.pallas import tpu as pltpu

_IN_F = 64
_OUT_F = 64
_ALPHA = 0.1
_K = 4
_T = 1024
_R = 256
_NEG = -9e15


def _gat_block_kernel(h_ref, hr_ref, w_ref, a_ref, o_ref):
    j = pl.program_id(1)

    X = jnp.dot(h_ref[0], w_ref[...], preferred_element_type=jnp.float32)
    Xr = jnp.dot(hr_ref[0], w_ref[...], preferred_element_type=jnp.float32)
    avec = a_ref[0, :]
    w2 = jnp.sum(X * avec[None, _OUT_F:], axis=1)
    w1r = jnp.sum(Xr * avec[None, :_OUT_F], axis=1)

    inner = 2.0 * jax.lax.dot_general(
        Xr, X, (((1,), (1,)), ((), ())), preferred_element_type=jnp.float32
    )
    xx = jnp.sum(X * X, axis=1)
    xxr = jnp.sum(Xr * Xr, axis=1)
    pd = -(xxr[:, None] - inner + xx[None, :])

    col = jax.lax.broadcasted_iota(jnp.int32, (_R, _T), 1)
    row = jax.lax.broadcasted_iota(jnp.int32, (_R, _T), 0) + j * _R

    work = pd
    sel = jnp.zeros((_R, _T), dtype=jnp.bool_)
    for _ in range(_K):
        m = jnp.max(work, axis=1, keepdims=True)
        cand = jnp.where(work == m, col, _T)
        first = jnp.min(cand, axis=1, keepdims=True)
        onehot = col == first
        sel = sel | onehot
        work = jnp.where(onehot, -1e30, work)

    adj = sel | (col == row)

    e = w1r[:, None] + w2[None, :]
    e = jnp.where(e >= 0, e, _ALPHA * e)
    logits = jnp.where(adj, e, _NEG)
    mx = jnp.max(logits, axis=1, keepdims=True)
    p = jnp.exp(logits - mx)
    att = p / jnp.sum(p, axis=1, keepdims=True)

    out = jnp.dot(att, X, preferred_element_type=jnp.float32)
    o_ref[0] = jnp.where(out > 0.0, out, jnp.exp(out) - 1.0)


def _run_blocks(hblk, W, avec):
    nblk = hblk.shape[0]
    grid = (nblk, _T // _R)
    return pl.pallas_call(
        _gat_block_kernel,
        grid=grid,
        in_specs=[
            pl.BlockSpec((1, _T, _IN_F), lambda b, j: (b, 0, 0)),
            pl.BlockSpec((1, _R, _IN_F), lambda b, j: (b, j, 0)),
            pl.BlockSpec((_IN_F, _OUT_F), lambda b, j: (0, 0)),
            pl.BlockSpec((1, 2 * _OUT_F), lambda b, j: (0, 0)),
        ],
        out_specs=pl.BlockSpec((1, _R, _OUT_F), lambda b, j: (b, j, 0)),
        out_shape=jax.ShapeDtypeStruct((nblk, _T, _OUT_F), jnp.float32),
        compiler_params=pltpu.CompilerParams(
            dimension_semantics=("parallel", "arbitrary"),
        ),
    )(hblk, hblk, W, avec)


@jax.jit
def kernel(x_a, x_v, W, a):
    bs, t, _ = x_a.shape
    h = jnp.concatenate([x_a, x_v], axis=1).reshape(bs * 2, t, _IN_F)
    avec = a.reshape(1, 2 * _OUT_F)
    out = _run_blocks(h, W, avec)
    out = out.reshape(bs, 2, t, _OUT_F)
    return (out[:, 0], out[:, 1])


# speedup vs baseline: 21.9703x; 1.5095x over previous
import jax
import jax.numpy as jnp
from jax.experimental import pallas as pl
from jax.experimental.pallas import tpu as pltpu

_IN_F = 64
_OUT_F = 64
_ALPHA = 0.1
_K = 4
_T = 1024


def _gat_block_kernel(h_ref, w_ref, a_ref, o_ref):
    X = jnp.dot(h_ref[0], w_ref[...], preferred_element_type=jnp.float32)
    avec = a_ref[0, :]
    w1 = jnp.sum(X * avec[None, :_OUT_F], axis=1)
    w2 = jnp.sum(X * avec[None, _OUT_F:], axis=1)

    inner = 2.0 * jax.lax.dot_general(
        X, X, (((1,), (1,)), ((), ())), preferred_element_type=jnp.float32
    )
    xx = jnp.sum(X * X, axis=1)
    pd = inner - xx[:, None] - xx[None, :]

    col = jax.lax.broadcasted_iota(jnp.int32, (_T, _T), 1)

    work = pd
    for _ in range(_K):
        m = jnp.max(work, axis=1, keepdims=True)
        cand = jnp.where(work == m, col, _T)
        first = jnp.min(cand, axis=1, keepdims=True)
        work = jnp.where(col == first, -1e30, work)

    row = jax.lax.broadcasted_iota(jnp.int32, (_T, _T), 0)
    adj = (work < -5e29) | (col == row)

    s = w1[:, None] + w2[None, :]
    e = jnp.maximum(s, _ALPHA * s)
    p = jnp.where(adj, jnp.exp(e), 0.0)
    z = jnp.sum(p, axis=1, keepdims=True)

    out = jnp.dot(p, X, preferred_element_type=jnp.float32) / z
    o_ref[0] = jnp.where(out > 0.0, out, jnp.exp(out) - 1.0)


def _run_blocks(hblk, W, avec):
    nblk = hblk.shape[0]
    return pl.pallas_call(
        _gat_block_kernel,
        grid=(nblk,),
        in_specs=[
            pl.BlockSpec((1, _T, _IN_F), lambda b: (b, 0, 0)),
            pl.BlockSpec((_IN_F, _OUT_F), lambda b: (0, 0)),
            pl.BlockSpec((1, 2 * _OUT_F), lambda b: (0, 0)),
        ],
        out_specs=pl.BlockSpec((1, _T, _OUT_F), lambda b: (b, 0, 0)),
        out_shape=jax.ShapeDtypeStruct((nblk, _T, _OUT_F), jnp.float32),
        compiler_params=pltpu.CompilerParams(
            dimension_semantics=("parallel",),
        ),
    )(hblk, W, avec)


@jax.jit
def kernel(x_a, x_v, W, a):
    bs, t, _ = x_a.shape
    h = jnp.concatenate([x_a, x_v], axis=1).reshape(bs * 2, t, _IN_F)
    avec = a.reshape(1, 2 * _OUT_F)
    out = _run_blocks(h, W, avec)
    out = out.reshape(bs, 2, t, _OUT_F)
    return (out[:, 0], out[:, 1])


# speedup vs baseline: 23.4639x; 1.0680x over previous
import jax
import jax.numpy as jnp
from jax.experimental import pallas as pl
from jax.experimental.pallas import tpu as pltpu

_IN_F = 64
_OUT_F = 64
_ALPHA = 0.1
_K = 4
_T = 1024


def _gat_block_kernel(h_ref, w_ref, a_ref, o_ref):
    X = jnp.dot(h_ref[0], w_ref[...], preferred_element_type=jnp.float32)
    a1 = a_ref[:, :_OUT_F]
    a2 = a_ref[:, _OUT_F:]
    w1 = jax.lax.dot_general(
        X, a1, (((1,), (1,)), ((), ())), preferred_element_type=jnp.float32
    )
    w2r = jax.lax.dot_general(
        a2, X, (((1,), (1,)), ((), ())), preferred_element_type=jnp.float32
    )

    inner = 2.0 * jax.lax.dot_general(
        X, X, (((1,), (1,)), ((), ())), preferred_element_type=jnp.float32
    )
    xx = jnp.sum(X * X, axis=1)
    pd = inner - xx[:, None] - xx[None, :]

    col = jax.lax.broadcasted_iota(jnp.int32, (_T, _T), 1)

    work = pd
    for _ in range(_K):
        m = jnp.max(work, axis=1, keepdims=True)
        cand = jnp.where(work == m, col, _T)
        first = jnp.min(cand, axis=1, keepdims=True)
        work = jnp.where(col == first, -1e30, work)

    row = jax.lax.broadcasted_iota(jnp.int32, (_T, _T), 0)
    adj = (work < -5e29) | (col == row)

    s = w1 + w2r
    e = jnp.maximum(s, _ALPHA * s)
    p = jnp.where(adj, jnp.exp(e), 0.0)

    xz = jnp.concatenate([X, jnp.ones((_T, 1), jnp.float32)], axis=1)
    num = jnp.dot(p, xz, preferred_element_type=jnp.float32)
    out = num[:, :_OUT_F] / num[:, _OUT_F:]
    o_ref[0] = jnp.where(out > 0.0, out, jnp.exp(out) - 1.0)


def _run_blocks(hblk, W, avec):
    nblk = hblk.shape[0]
    return pl.pallas_call(
        _gat_block_kernel,
        grid=(nblk,),
        in_specs=[
            pl.BlockSpec((1, _T, _IN_F), lambda b: (b, 0, 0)),
            pl.BlockSpec((_IN_F, _OUT_F), lambda b: (0, 0)),
            pl.BlockSpec((1, 2 * _OUT_F), lambda b: (0, 0)),
        ],
        out_specs=pl.BlockSpec((1, _T, _OUT_F), lambda b: (b, 0, 0)),
        out_shape=jax.ShapeDtypeStruct((nblk, _T, _OUT_F), jnp.float32),
        compiler_params=pltpu.CompilerParams(
            dimension_semantics=("parallel",),
        ),
    )(hblk, W, avec)


@jax.jit
def kernel(x_a, x_v, W, a):
    bs, t, _ = x_a.shape
    h = jnp.concatenate([x_a, x_v], axis=1).reshape(bs * 2, t, _IN_F)
    avec = a.reshape(1, 2 * _OUT_F)
    out = _run_blocks(h, W, avec)
    out = out.reshape(bs, 2, t, _OUT_F)
    return (out[:, 0], out[:, 1])
